# baseline (speedup 1.0000x reference)
import functools

import jax
import jax.numpy as jnp
from jax import lax
from jax.experimental import pallas as pl
from jax.experimental.pallas import tpu as pltpu
from jax.experimental.pallas import tpu_sc as plsc

NC = 2
NS = 16
L = 16
NW = NC * NS
ROWS = 128
COLS = 32768
SC_ROWS = 48
TC_ROWS = ROWS - SC_ROWS
TCRB = 16
SLAB = SC_ROWS * COLS // NW
CH = 8192
NCHUNK = SLAB // CH
VPC = CH // L
U = 4
BIG = 2**31 - 1
NEG = float("-inf")

_MESH = plsc.VectorSubcoreMesh(core_axis_name="c", subcore_axis_name="s",
                               num_cores=NC, num_subcores=NS)

_GDN = lax.GatherDimensionNumbers(
    offset_dims=(), collapsed_slice_dims=(0,), start_index_map=(0,))


def _shuffle(v, idx):
    return lax.gather(v, idx[:, None], _GDN, (1,),
                      mode=lax.GatherScatterMode.PROMISE_IN_BOUNDS)


@functools.partial(
    pl.kernel,
    out_type=(
        jax.ShapeDtypeStruct((NW, L), jnp.float32),
        jax.ShapeDtypeStruct((NW, L), jnp.int32),
    ),
    mesh=_MESH,
    scratch_types=[
        pltpu.VMEM((CH,), jnp.float32),
        pltpu.VMEM((CH,), jnp.float32),
        pltpu.VMEM((CH,), jnp.float32),
        pltpu.VMEM((CH,), jnp.float32),
        pltpu.VMEM((CH,), jnp.float32),
        pltpu.VMEM((CH,), jnp.float32),
        pltpu.VMEM((CH,), jnp.float32),
        pltpu.VMEM((CH,), jnp.float32),
        pltpu.VMEM((L,), jnp.float32),
        pltpu.VMEM((L,), jnp.int32),
        pltpu.SemaphoreType.DMA,
        pltpu.SemaphoreType.DMA,
        pltpu.SemaphoreType.DMA,
        pltpu.SemaphoreType.DMA,
        pltpu.SemaphoreType.DMA,
        pltpu.SemaphoreType.DMA,
        pltpu.SemaphoreType.DMA,
        pltpu.SemaphoreType.DMA,
    ],
)
def _sc_scan(x_hbm, vals_hbm, idxs_hbm, buf0, buf1, buf2, buf3,
             buf4, buf5, buf6, buf7, stage_v, stage_i,
             sem0, sem1, sem2, sem3, sem4, sem5, sem6, sem7):
    wid = lax.axis_index("s") * NC + lax.axis_index("c")
    base = wid * SLAB
    bufs = (buf0, buf1, buf2, buf3, buf4, buf5, buf6, buf7)
    sems = (sem0, sem1, sem2, sem3, sem4, sem5, sem6, sem7)
    lane = lax.iota(jnp.int32, L)

    for c in range(NCHUNK):
        off = base + c * CH
        row = off // COLS
        col = pl.multiple_of(off % COLS, CH)
        pltpu.async_copy(x_hbm.at[row, pl.ds(col, CH)], bufs[c], sems[c])

    accs = tuple(jnp.full((L,), NEG, jnp.float32) for _ in range(U))
    poss = tuple(jnp.full((L,), 0, jnp.int32) for _ in range(U))
    for c in range(NCHUNK):
        pltpu.make_async_copy(x_hbm.at[0, pl.ds(0, CH)], bufs[c],
                              sems[c]).wait()
        buf = bufs[c]

        @plsc.parallel_loop(0, VPC, step=U, unroll=2, carry=(accs, poss))
        def res(i, carry, buf=buf, c=c):
            a, p = carry
            gvec = jnp.full((L,), i + c * VPC)
            na, np_ = [], []
            for u in range(U):
                v = buf[pl.ds((i + u) * L, L)]
                gt = v > a[u]
                na.append(jnp.where(gt, v, a[u]))
                np_.append(jnp.where(gt, gvec, p[u]))
            return (tuple(na), tuple(np_))

        accs, poss = res

    bv = accs[0]
    bi = base + (poss[0] + 0) * L + lane
    for u in range(1, U):
        fv = accs[u]
        fi = base + (poss[u] + u) * L + lane
        better = (fv > bv) | ((fv == bv) & (fi < bi))
        bv = jnp.where(better, fv, bv)
        bi = jnp.where(better, fi, bi)

    for sh in (8, 4, 2, 1):
        idx2 = lane ^ sh
        v2 = _shuffle(bv, idx2)
        i2 = _shuffle(bi, idx2)
        better = (v2 > bv) | ((v2 == bv) & (i2 < bi))
        bv = jnp.where(better, v2, bv)
        bi = jnp.where(better, i2, bi)

    stage_v[...] = bv
    stage_i[...] = bi
    pltpu.sync_copy(stage_v, vals_hbm.at[wid])
    pltpu.sync_copy(stage_i, idxs_hbm.at[wid])


def _tc_body(x_ref, val_ref, idx_ref, rmax_ref, ridx_ref):
    b = pl.program_id(0)

    @pl.when(b == 0)
    def _init():
        rmax_ref[0] = -jnp.inf
        ridx_ref[0] = jnp.int32(BIG)

    xb = x_ref[...]
    m = jnp.max(xb)

    @pl.when(m >= rmax_ref[0])
    def _update():
        rows = lax.broadcasted_iota(jnp.int32, (TCRB, COLS), 0)
        cols = lax.broadcasted_iota(jnp.int32, (TCRB, COLS), 1)
        flat = (rows + SC_ROWS + b * TCRB) * COLS + cols
        cand = jnp.min(jnp.where(xb == m, flat, jnp.int32(BIG)))
        old_m = rmax_ref[0]
        old_i = ridx_ref[0]
        better = (m > old_m) | (cand < old_i)
        ridx_ref[0] = jnp.where(better, cand, old_i)
        rmax_ref[0] = jnp.where(m > old_m, m, old_m)

    @pl.when(b == pl.num_programs(0) - 1)
    def _fin():
        val_ref[0] = rmax_ref[0]
        idx_ref[0] = ridx_ref[0]


def _merge_body(vals_ref, idxs_ref, tcv_ref, tci_ref, out_ref):
    v = vals_ref[...]
    ix = idxs_ref[...]
    m = jnp.max(v)
    cand = jnp.min(jnp.where(v == m, ix, jnp.int32(BIG)))
    tv = tcv_ref[0]
    ti = tci_ref[0]
    better_tc = (tv > m) | ((tv == m) & (ti < cand))
    out_ref[0] = jnp.where(better_tc, ti, cand)


def kernel(x):
    tcv, tci = pl.pallas_call(
        _tc_body,
        grid=(TC_ROWS // TCRB,),
        in_specs=[pl.BlockSpec((TCRB, COLS),
                               lambda b: (b + SC_ROWS // TCRB, 0))],
        out_specs=(
            pl.BlockSpec(memory_space=pltpu.SMEM),
            pl.BlockSpec(memory_space=pltpu.SMEM),
        ),
        out_shape=(
            jax.ShapeDtypeStruct((1,), jnp.float32),
            jax.ShapeDtypeStruct((1,), jnp.int32),
        ),
        scratch_shapes=[
            pltpu.SMEM((1,), jnp.float32),
            pltpu.SMEM((1,), jnp.int32),
        ],
    )(x)
    vals, idxs = _sc_scan(x)
    merged = pl.pallas_call(
        _merge_body,
        in_specs=[
            pl.BlockSpec((NW, L), lambda: (0, 0)),
            pl.BlockSpec((NW, L), lambda: (0, 0)),
            pl.BlockSpec(memory_space=pltpu.SMEM),
            pl.BlockSpec(memory_space=pltpu.SMEM),
        ],
        out_specs=pl.BlockSpec(memory_space=pltpu.SMEM),
        out_shape=jax.ShapeDtypeStruct((1,), jnp.int32),
    )(vals, idxs, tcv, tci)
    return merged[0].astype(jnp.int64)
